# baseline (speedup 1.0000x reference)
import functools

import jax
import jax.numpy as jnp
from jax import lax
from jax.experimental import pallas as pl
from jax.experimental.pallas import tpu as pltpu
from jax.experimental.pallas import tpu_sc as plsc

N = 10000
E = 320000
D_IN = 128
NH = 32
HEADS = 4
D_OUT = 128

TW = 144
UW = 16
NC = 2
NS = 16
NW = NC * NS
EPW = E // NW
B = 80
NCHUNK = -(-EPW // B)
EPW_PAD = NCHUNK * B

NG = B // 16
NP = 10240
ROWS_PT = NP // NS


def _pre_body(x_ref, w1_ref, b1_ref, wl_ref, u16_ref, c16_ref,
              h_ref, t_ref, tu_ref):
    h = jax.nn.relu(jnp.dot(x_ref[...], w1_ref[...],
                            preferred_element_type=jnp.float32) + b1_ref[...])
    ujc = jnp.dot(h, u16_ref[...], preferred_element_type=jnp.float32) + c16_ref[...]
    m = jnp.dot(h, wl_ref[...], preferred_element_type=jnp.float32)
    h_ref[...] = h
    t_ref[...] = jnp.concatenate(
        [m, ujc - jnp.max(ujc, axis=1, keepdims=True)], axis=1)
    u0 = ujc - c16_ref[...]
    tu_ref[...] = u0 - jnp.min(u0, axis=1, keepdims=True)


def _mid_body(acc_ref, deg_ref, h_ref, wls_ref, bb_ref,
              wln_ref, u16n_ref, c16n_ref,
              hn_ref, tn_ref, tun_ref):
    dinv16 = 1.0 / (deg_ref[0, :N] + deg_ref[1, :N] + 1.0)
    dinv32 = jnp.concatenate([dinv16, dinv16], axis=1)
    h = h_ref[...]
    seg = acc_ref[0, :N] + acc_ref[1, :N] + jnp.dot(h, wls_ref[...],
                                            preferred_element_type=jnp.float32)
    hn = jax.nn.relu(seg * dinv32 + bb_ref[...])
    ujc = jnp.dot(hn, u16n_ref[...], preferred_element_type=jnp.float32) + c16n_ref[...]
    m = jnp.dot(hn, wln_ref[...], preferred_element_type=jnp.float32)
    hn_ref[...] = hn
    tn_ref[...] = jnp.concatenate(
        [m, ujc - jnp.max(ujc, axis=1, keepdims=True)], axis=1)
    u0 = ujc - c16n_ref[...]
    tun_ref[...] = u0 - jnp.min(u0, axis=1, keepdims=True)


def _post_body(acc_ref, deg_ref, h_ref, wls_ref, bb_ref,
               w2_ref, b2_ref, w3_ref, b3_ref, out_ref):
    dinv16 = 1.0 / (deg_ref[0, :N] + deg_ref[1, :N] + 1.0)
    dinv32 = jnp.concatenate([dinv16, dinv16], axis=1)
    h = h_ref[...]
    seg = acc_ref[0, :N] + acc_ref[1, :N] + jnp.dot(h, wls_ref[...],
                                            preferred_element_type=jnp.float32)
    hn = jax.nn.relu(seg * dinv32 + bb_ref[...])
    h2 = jax.nn.relu(jnp.dot(hn, w2_ref[...],
                             preferred_element_type=jnp.float32) + b2_ref[...])
    out_ref[...] = jnp.dot(h2, w3_ref[...],
                           preferred_element_type=jnp.float32) + b3_ref[...]


_f32 = jnp.float32

_tc_pre = pl.pallas_call(
    _pre_body,
    out_shape=[jax.ShapeDtypeStruct((N, NH), _f32),
               jax.ShapeDtypeStruct((N, TW), _f32),
               jax.ShapeDtypeStruct((N, UW), _f32)],
)

_tc_mid = pl.pallas_call(
    _mid_body,
    out_shape=[jax.ShapeDtypeStruct((N, NH), _f32),
               jax.ShapeDtypeStruct((N, TW), _f32),
               jax.ShapeDtypeStruct((N, UW), _f32)],
)

_tc_post = pl.pallas_call(
    _post_body,
    out_shape=jax.ShapeDtypeStruct((N, D_OUT), _f32),
)


def _sc_body(with_deg, *refs):
    if with_deg:
        (t_hbm, tu_hbm, src_hbm, dst_hbm, z32_hbm, z16_hbm, ones_hbm,
         acc_out, deg_out,
         sidx0, sidx1, didx0, didx1, dscat0, dscat1,
         rows0, rows1, udst0, udst1, msg0, msg1, ones_v, stage32, stage16,
         acc_sh, deg_sh,
         si0, si1, di0, di1, sg0, sg1, su0, su1, sc0, sc1, sd0, sd1) = refs
    else:
        (t_hbm, tu_hbm, src_hbm, dst_hbm, z32_hbm,
         acc_out,
         sidx0, sidx1, didx0, didx1, dscat0, dscat1,
         rows0, rows1, udst0, udst1, msg0, msg1, stage32,
         acc_sh,
         si0, si1, di0, di1, sg0, sg1, su0, su1, sc0, sc1) = refs

    sidx = (sidx0, sidx1)
    didx = (didx0, didx1)
    dscat = (dscat0, dscat1)
    rows = (rows0, rows1)
    udst = (udst0, udst1)
    msg = (msg0, msg1)
    si = (si0, si1)
    di = (di0, di1)
    sg = (sg0, sg1)
    su = (su0, su1)
    sc = (sc0, sc1)
    if with_deg:
        sd = (sd0, sd1)

    c = lax.axis_index("c")
    s = lax.axis_index("s")
    wid = s * NC + c
    r0 = s * ROWS_PT

    pltpu.sync_copy(z32_hbm.at[pl.ds(r0, ROWS_PT)], stage32)
    pltpu.sync_copy(stage32, acc_sh.at[pl.ds(r0, ROWS_PT)])
    if with_deg:
        pltpu.sync_copy(z16_hbm.at[pl.ds(r0, ROWS_PT)], stage16)
        pltpu.sync_copy(stage16, deg_sh.at[pl.ds(r0, ROWS_PT)])
        pltpu.sync_copy(ones_hbm, ones_v)
    plsc.subcore_barrier()

    def idx_start(j, b):
        base = wid * EPW_PAD + j * B
        pltpu.async_copy(src_hbm.at[pl.ds(base, B)], sidx[b], si[b])
        pltpu.async_copy(dst_hbm.at[pl.ds(base, B)], didx[b], di[b])

    def idx_wait(j, b):
        base = wid * EPW_PAD + j * B
        pltpu.make_async_copy(src_hbm.at[pl.ds(base, B)], sidx[b], si[b]).wait()
        pltpu.make_async_copy(dst_hbm.at[pl.ds(base, B)], didx[b], di[b]).wait()

    def gather_start(b):
        pltpu.async_copy(t_hbm.at[sidx[b]], rows[b], sg[b])
        pltpu.async_copy(tu_hbm.at[didx[b]], udst[b], su[b])

    def gather_wait(b):
        pltpu.make_async_copy(t_hbm.at[sidx[b]], rows[b], sg[b]).wait()
        pltpu.make_async_copy(tu_hbm.at[didx[b]], udst[b], su[b]).wait()

    def scatter_start(b):
        pltpu.async_copy(msg[b], acc_sh.at[dscat[b]], sc[b], add=True)
        if with_deg:
            pltpu.async_copy(ones_v, deg_sh.at[dscat[b]], sd[b], add=True)

    def scatter_wait(b):
        pltpu.make_async_copy(msg[b], acc_sh.at[dscat[b]], sc[b]).wait()
        if with_deg:
            pltpu.make_async_copy(ones_v, deg_sh.at[dscat[b]], sd[b]).wait()

    def save_didx(b):
        for g in range(B // 16):
            dscat[b][pl.ds(16 * g, 16)] = didx[b][pl.ds(16 * g, 16)]

    _io = lax.iota(jnp.int32, 16)
    _rot = [(_io & ~3) + ((_io + r) & 3) for r in (1, 2, 3)]
    _bidx = [jnp.full((16,), h, jnp.int32) for h in range(HEADS)]

    def compute(b):
        rb = rows[b]
        ub = udst[b]
        mb = msg[b]

        def take(v, idx):
            return v.at[idx].get(mode='promise_in_bounds')

        def edge(eb, carry2):
            for d in range(8):
                e = eb * 8 + d
                uj = rb[e, pl.ds(128, 16)]
                ui = ub[e, pl.ds(0, 16)]
                w = jnp.exp(uj - ui)
                z2 = w + take(w, _rot[1])
                z = z2 + take(z2, _rot[0])
                q = w / z
                qb = [take(q, _bidx[h]) for h in range(HEADS)]
                lo = qb[0] * rb[e, pl.ds(0, 16)]
                hi = qb[0] * rb[e, pl.ds(16, 16)]
                for h in range(1, HEADS):
                    lo = lo + qb[h] * rb[e, pl.ds(32 * h, 16)]
                    hi = hi + qb[h] * rb[e, pl.ds(32 * h + 16, 16)]
                mb[e, pl.ds(0, 16)] = lo
                mb[e, pl.ds(16, 16)] = hi
            return carry2

        lax.fori_loop(0, B // 8, edge, 0)

    idx_start(0, 0)
    idx_wait(0, 0)
    gather_start(0)
    idx_start(1, 1)
    idx_wait(1, 1)
    gather_start(1)
    gather_wait(0)
    save_didx(0)
    idx_start(2, 0)
    compute(0)
    scatter_start(0)
    idx_wait(2, 0)
    gather_start(0)
    gather_wait(1)
    save_didx(1)
    idx_start(3, 1)
    compute(1)
    scatter_start(1)

    def steady(k, carry):
        t0 = 4 + 2 * k
        for m in (0, 1):
            t = t0 + m
            idx_wait(t - 1, 1 - m)
            gather_start(1 - m)
            gather_wait(m)
            scatter_wait(m)
            save_didx(m)
            idx_start(t, m)
            compute(m)
            scatter_start(m)
        return carry

    lax.fori_loop(0, (NCHUNK - 4) // 2, steady, 0)

    if (NCHUNK - 4) % 2 == 1:
        t = NCHUNK - 1
        m = t % 2
        idx_wait(t - 1, 1 - m)
        gather_start(1 - m)
        gather_wait(m)
        scatter_wait(m)
        save_didx(m)
        idx_start(t, m)
        compute(m)
        scatter_start(m)
    pL = (NCHUNK - 1) % 2
    pS = 1 - pL
    idx_wait(NCHUNK - 1, pL)
    gather_start(pL)
    gather_wait(pS)
    scatter_wait(pS)
    save_didx(pS)
    compute(pS)
    scatter_start(pS)
    gather_wait(pL)
    scatter_wait(pL)
    save_didx(pL)
    compute(pL)
    scatter_start(pL)
    scatter_wait(pS)
    scatter_wait(pL)

    plsc.subcore_barrier()

    pltpu.sync_copy(acc_sh.at[pl.ds(r0, ROWS_PT)], stage32)
    pltpu.sync_copy(stage32, acc_out.at[c, pl.ds(r0, ROWS_PT)])
    if with_deg:
        pltpu.sync_copy(deg_sh.at[pl.ds(r0, ROWS_PT)], stage16)
        pltpu.sync_copy(stage16, deg_out.at[c, pl.ds(r0, ROWS_PT)])


_SC_MESH = plsc.VectorSubcoreMesh(core_axis_name="c", subcore_axis_name="s")
_SC_PARAMS = pltpu.CompilerParams(use_tc_tiling_on_sc=False,
                                  needs_layout_passes=False)

_sc_layer0 = pl.kernel(
    functools.partial(_sc_body, True),
    out_type=[jax.ShapeDtypeStruct((NC, NP, NH), _f32),
              jax.ShapeDtypeStruct((NC, NP, UW), _f32)],
    mesh=_SC_MESH,
    scratch_types=(
        [pltpu.VMEM((B,), jnp.int32)] * 6
        + [pltpu.VMEM((B, TW), _f32)] * 2
        + [pltpu.VMEM((B, UW), _f32)] * 2
        + [pltpu.VMEM((B, NH), _f32)] * 2
        + [pltpu.VMEM((B, UW), _f32)]
        + [pltpu.VMEM((ROWS_PT, NH), _f32),
           pltpu.VMEM((ROWS_PT, UW), _f32),
           pltpu.VMEM_SHARED((NP, NH), _f32),
           pltpu.VMEM_SHARED((NP, UW), _f32)]
        + [pltpu.SemaphoreType.DMA] * 12
    ),
    compiler_params=_SC_PARAMS,
)

_sc_layer = pl.kernel(
    functools.partial(_sc_body, False),
    out_type=jax.ShapeDtypeStruct((NC, NP, NH), _f32),
    mesh=_SC_MESH,
    scratch_types=(
        [pltpu.VMEM((B,), jnp.int32)] * 6
        + [pltpu.VMEM((B, TW), _f32)] * 2
        + [pltpu.VMEM((B, UW), _f32)] * 2
        + [pltpu.VMEM((B, NH), _f32)] * 2
        + [pltpu.VMEM((ROWS_PT, NH), _f32),
           pltpu.VMEM_SHARED((NP, NH), _f32)]
        + [pltpu.SemaphoreType.DMA] * 10
    ),
    compiler_params=_SC_PARAMS,
)


def kernel(x, edge_index, W1, b1, W2, b2, W3, b3,
           Wl0, U0, c0, bb0, Wl1, U1, c1, bb1, Wl2, U2, c2, bb2):
    f32 = jnp.float32

    def pad_u(u):
        return jnp.tile(u.astype(f32), (1, UW // HEADS))

    def pad_c(c):
        return jnp.tile(c.astype(f32), UW // HEADS).reshape(1, UW)

    def wls(wl, c):
        wc = jax.nn.softmax(c.astype(f32))
        return (wl.astype(f32).reshape(NH, HEADS, NH) * wc[None, :, None]).sum(axis=1)

    pad = EPW_PAD - EPW
    srcs = jnp.concatenate(
        [edge_index[0].astype(jnp.int32).reshape(NW, EPW),
         jnp.zeros((NW, pad), jnp.int32)], axis=1).reshape(-1)
    dsts = jnp.concatenate(
        [edge_index[1].astype(jnp.int32).reshape(NW, EPW),
         jnp.full((NW, pad), N, jnp.int32)], axis=1).reshape(-1)
    z32 = jnp.zeros((NP, NH), f32)
    z16 = jnp.zeros((NP, UW), f32)
    ones = jnp.ones((B, UW), f32)

    b1r = b1.reshape(1, NH).astype(f32)
    b2r = b2.reshape(1, NH // 2).astype(f32)
    b3r = b3.reshape(1, D_OUT).astype(f32)
    bb = [b.reshape(1, NH).astype(f32) for b in (bb0, bb1, bb2)]
    wl = [w.astype(f32) for w in (Wl0, Wl1, Wl2)]
    u16 = [pad_u(u) for u in (U0, U1, U2)]
    c16 = [pad_c(c) for c in (c0, c1, c2)]
    wlss = [wls(w, c) for w, c in ((Wl0, c0), (Wl1, c1), (Wl2, c2))]

    h0, t0, tu0 = _tc_pre(x.astype(f32), W1.astype(f32), b1r,
                          wl[0], u16[0], c16[0])
    acc0, deg = _sc_layer0(t0, tu0, srcs, dsts, z32, z16, ones)
    h1, t1, tu1 = _tc_mid(acc0, deg, h0, wlss[0], bb[0],
                          wl[1], u16[1], c16[1])
    acc1 = _sc_layer(t1, tu1, srcs, dsts, z32)
    h2, t2, tu2 = _tc_mid(acc1, deg, h1, wlss[1], bb[1],
                          wl[2], u16[2], c16[2])
    acc2 = _sc_layer(t2, tu2, srcs, dsts, z32)
    return _tc_post(acc2, deg, h2, wlss[2], bb[2],
                    W2.astype(f32), b2r, W3.astype(f32), b3r)


# speedup vs baseline: 2.0600x; 2.0600x over previous
import functools

import jax
import jax.numpy as jnp
from jax import lax
from jax.experimental import pallas as pl
from jax.experimental.pallas import tpu as pltpu
from jax.experimental.pallas import tpu_sc as plsc

N = 10000
E = 320000
D_IN = 128
NH = 32
HEADS = 4
D_OUT = 128

TW = 144
UW = 16
NC = 2
NS = 16
NW = NC * NS
EPW = E // NW
B = 80
NCHUNK = -(-EPW // B)
EPW_PAD = NCHUNK * B

NG = B // 16
NP = 10240
ROWS_PT = NP // NS


def _pre_body(x_ref, w1_ref, b1_ref, wl_ref, u16_ref, c16_ref,
              h_ref, t_ref, tu_ref):
    h = jax.nn.relu(jnp.dot(x_ref[...], w1_ref[...],
                            preferred_element_type=jnp.float32) + b1_ref[...])
    ujc = jnp.dot(h, u16_ref[...], preferred_element_type=jnp.float32) + c16_ref[...]
    m = jnp.dot(h, wl_ref[...], preferred_element_type=jnp.float32)
    h_ref[...] = h
    t_ref[...] = jnp.concatenate(
        [m, jnp.exp(ujc - jnp.max(ujc, axis=1, keepdims=True))], axis=1)
    u0 = ujc - c16_ref[...]
    tu_ref[...] = jnp.exp(jnp.min(u0, axis=1, keepdims=True) - u0)


def _mid_body(acc_ref, deg_ref, h_ref, wls_ref, bb_ref,
              wln_ref, u16n_ref, c16n_ref,
              hn_ref, tn_ref, tun_ref):
    dinv16 = 1.0 / (deg_ref[0, :N] + deg_ref[1, :N] + 1.0)
    dinv32 = jnp.concatenate([dinv16, dinv16], axis=1)
    h = h_ref[...]
    seg = acc_ref[0, :N] + acc_ref[1, :N] + jnp.dot(h, wls_ref[...],
                                            preferred_element_type=jnp.float32)
    hn = jax.nn.relu(seg * dinv32 + bb_ref[...])
    ujc = jnp.dot(hn, u16n_ref[...], preferred_element_type=jnp.float32) + c16n_ref[...]
    m = jnp.dot(hn, wln_ref[...], preferred_element_type=jnp.float32)
    hn_ref[...] = hn
    tn_ref[...] = jnp.concatenate(
        [m, jnp.exp(ujc - jnp.max(ujc, axis=1, keepdims=True))], axis=1)
    u0 = ujc - c16n_ref[...]
    tun_ref[...] = jnp.exp(jnp.min(u0, axis=1, keepdims=True) - u0)


def _post_body(acc_ref, deg_ref, h_ref, wls_ref, bb_ref,
               w2_ref, b2_ref, w3_ref, b3_ref, out_ref):
    dinv16 = 1.0 / (deg_ref[0, :N] + deg_ref[1, :N] + 1.0)
    dinv32 = jnp.concatenate([dinv16, dinv16], axis=1)
    h = h_ref[...]
    seg = acc_ref[0, :N] + acc_ref[1, :N] + jnp.dot(h, wls_ref[...],
                                            preferred_element_type=jnp.float32)
    hn = jax.nn.relu(seg * dinv32 + bb_ref[...])
    h2 = jax.nn.relu(jnp.dot(hn, w2_ref[...],
                             preferred_element_type=jnp.float32) + b2_ref[...])
    out_ref[...] = jnp.dot(h2, w3_ref[...],
                           preferred_element_type=jnp.float32) + b3_ref[...]


_f32 = jnp.float32

_tc_pre = pl.pallas_call(
    _pre_body,
    out_shape=[jax.ShapeDtypeStruct((N, NH), _f32),
               jax.ShapeDtypeStruct((N, TW), _f32),
               jax.ShapeDtypeStruct((N, UW), _f32)],
)

_tc_mid = pl.pallas_call(
    _mid_body,
    out_shape=[jax.ShapeDtypeStruct((N, NH), _f32),
               jax.ShapeDtypeStruct((N, TW), _f32),
               jax.ShapeDtypeStruct((N, UW), _f32)],
)

_tc_post = pl.pallas_call(
    _post_body,
    out_shape=jax.ShapeDtypeStruct((N, D_OUT), _f32),
)


def _sc_body(with_deg, *refs):
    if with_deg:
        (t_hbm, tu_hbm, src_hbm, dst_hbm, z32_hbm, z16_hbm, ones_hbm,
         acc_out, deg_out,
         sidx0, sidx1, didx0, didx1, dscat0, dscat1,
         rows0, rows1, udst0, udst1, msg0, msg1, ones_v, stage32, stage16,
         acc_sh, deg_sh,
         si0, si1, di0, di1, sg0, sg1, su0, su1, sc0, sc1, sd0, sd1) = refs
    else:
        (t_hbm, tu_hbm, src_hbm, dst_hbm, z32_hbm,
         acc_out,
         sidx0, sidx1, didx0, didx1, dscat0, dscat1,
         rows0, rows1, udst0, udst1, msg0, msg1, stage32,
         acc_sh,
         si0, si1, di0, di1, sg0, sg1, su0, su1, sc0, sc1) = refs

    sidx = (sidx0, sidx1)
    didx = (didx0, didx1)
    dscat = (dscat0, dscat1)
    rows = (rows0, rows1)
    udst = (udst0, udst1)
    msg = (msg0, msg1)
    si = (si0, si1)
    di = (di0, di1)
    sg = (sg0, sg1)
    su = (su0, su1)
    sc = (sc0, sc1)
    if with_deg:
        sd = (sd0, sd1)

    c = lax.axis_index("c")
    s = lax.axis_index("s")
    wid = s * NC + c
    r0 = s * ROWS_PT

    pltpu.sync_copy(z32_hbm.at[pl.ds(r0, ROWS_PT)], stage32)
    pltpu.sync_copy(stage32, acc_sh.at[pl.ds(r0, ROWS_PT)])
    if with_deg:
        pltpu.sync_copy(z16_hbm.at[pl.ds(r0, ROWS_PT)], stage16)
        pltpu.sync_copy(stage16, deg_sh.at[pl.ds(r0, ROWS_PT)])
        pltpu.sync_copy(ones_hbm, ones_v)
    plsc.subcore_barrier()

    def idx_start(j, b):
        base = wid * EPW_PAD + j * B
        pltpu.async_copy(src_hbm.at[pl.ds(base, B)], sidx[b], si[b])
        pltpu.async_copy(dst_hbm.at[pl.ds(base, B)], didx[b], di[b])

    def idx_wait(j, b):
        base = wid * EPW_PAD + j * B
        pltpu.make_async_copy(src_hbm.at[pl.ds(base, B)], sidx[b], si[b]).wait()
        pltpu.make_async_copy(dst_hbm.at[pl.ds(base, B)], didx[b], di[b]).wait()

    def gather_start(b):
        pltpu.async_copy(t_hbm.at[sidx[b]], rows[b], sg[b])
        pltpu.async_copy(tu_hbm.at[didx[b]], udst[b], su[b])

    def gather_wait(b):
        pltpu.make_async_copy(t_hbm.at[sidx[b]], rows[b], sg[b]).wait()
        pltpu.make_async_copy(tu_hbm.at[didx[b]], udst[b], su[b]).wait()

    def scatter_start(b):
        pltpu.async_copy(msg[b], acc_sh.at[dscat[b]], sc[b], add=True)
        if with_deg:
            pltpu.async_copy(ones_v, deg_sh.at[dscat[b]], sd[b], add=True)

    def scatter_wait(b):
        pltpu.make_async_copy(msg[b], acc_sh.at[dscat[b]], sc[b]).wait()
        if with_deg:
            pltpu.make_async_copy(ones_v, deg_sh.at[dscat[b]], sd[b]).wait()

    def save_didx(b):
        for g in range(B // 16):
            dscat[b][pl.ds(16 * g, 16)] = didx[b][pl.ds(16 * g, 16)]

    _io = lax.iota(jnp.int32, 16)
    _rot = [(_io & ~3) + ((_io + r) & 3) for r in (1, 2, 3)]
    _bidx = [jnp.full((16,), h, jnp.int32) for h in range(HEADS)]

    def compute(b):
        rb = rows[b]
        ub = udst[b]
        mb = msg[b]

        def take(v, idx):
            return v.at[idx].get(mode='promise_in_bounds')

        @plsc.parallel_loop(0, B, step=1, unroll=8)
        def edge(e):
            uj = rb[e, pl.ds(128, 16)]
            ui = ub[e, pl.ds(0, 16)]
            w = uj * ui
            z2 = w + take(w, _rot[1])
            z = z2 + take(z2, _rot[0])
            q = w / z
            qb = [take(q, _bidx[h]) for h in range(HEADS)]
            lo = qb[0] * rb[e, pl.ds(0, 16)]
            hi = qb[0] * rb[e, pl.ds(16, 16)]
            for h in range(1, HEADS):
                lo = lo + qb[h] * rb[e, pl.ds(32 * h, 16)]
                hi = hi + qb[h] * rb[e, pl.ds(32 * h + 16, 16)]
            mb[e, pl.ds(0, 16)] = lo
            mb[e, pl.ds(16, 16)] = hi

    idx_start(0, 0)
    idx_wait(0, 0)
    gather_start(0)
    idx_start(1, 1)
    idx_wait(1, 1)
    gather_start(1)
    gather_wait(0)
    save_didx(0)
    idx_start(2, 0)
    compute(0)
    scatter_start(0)
    idx_wait(2, 0)
    gather_start(0)
    gather_wait(1)
    save_didx(1)
    idx_start(3, 1)
    compute(1)
    scatter_start(1)

    def steady(k, carry):
        t0 = 4 + 2 * k
        for m in (0, 1):
            t = t0 + m
            idx_wait(t - 1, 1 - m)
            gather_start(1 - m)
            gather_wait(m)
            scatter_wait(m)
            save_didx(m)
            idx_start(t, m)
            compute(m)
            scatter_start(m)
        return carry

    lax.fori_loop(0, (NCHUNK - 4) // 2, steady, 0)

    if (NCHUNK - 4) % 2 == 1:
        t = NCHUNK - 1
        m = t % 2
        idx_wait(t - 1, 1 - m)
        gather_start(1 - m)
        gather_wait(m)
        scatter_wait(m)
        save_didx(m)
        idx_start(t, m)
        compute(m)
        scatter_start(m)
    pL = (NCHUNK - 1) % 2
    pS = 1 - pL
    idx_wait(NCHUNK - 1, pL)
    gather_start(pL)
    gather_wait(pS)
    scatter_wait(pS)
    save_didx(pS)
    compute(pS)
    scatter_start(pS)
    gather_wait(pL)
    scatter_wait(pL)
    save_didx(pL)
    compute(pL)
    scatter_start(pL)
    scatter_wait(pS)
    scatter_wait(pL)

    plsc.subcore_barrier()

    pltpu.sync_copy(acc_sh.at[pl.ds(r0, ROWS_PT)], stage32)
    pltpu.sync_copy(stage32, acc_out.at[c, pl.ds(r0, ROWS_PT)])
    if with_deg:
        pltpu.sync_copy(deg_sh.at[pl.ds(r0, ROWS_PT)], stage16)
        pltpu.sync_copy(stage16, deg_out.at[c, pl.ds(r0, ROWS_PT)])


_SC_MESH = plsc.VectorSubcoreMesh(core_axis_name="c", subcore_axis_name="s")
_SC_PARAMS = pltpu.CompilerParams(use_tc_tiling_on_sc=False,
                                  needs_layout_passes=False)

_sc_layer0 = pl.kernel(
    functools.partial(_sc_body, True),
    out_type=[jax.ShapeDtypeStruct((NC, NP, NH), _f32),
              jax.ShapeDtypeStruct((NC, NP, UW), _f32)],
    mesh=_SC_MESH,
    scratch_types=(
        [pltpu.VMEM((B,), jnp.int32)] * 6
        + [pltpu.VMEM((B, TW), _f32)] * 2
        + [pltpu.VMEM((B, UW), _f32)] * 2
        + [pltpu.VMEM((B, NH), _f32)] * 2
        + [pltpu.VMEM((B, UW), _f32)]
        + [pltpu.VMEM((ROWS_PT, NH), _f32),
           pltpu.VMEM((ROWS_PT, UW), _f32),
           pltpu.VMEM_SHARED((NP, NH), _f32),
           pltpu.VMEM_SHARED((NP, UW), _f32)]
        + [pltpu.SemaphoreType.DMA] * 12
    ),
    compiler_params=_SC_PARAMS,
)

_sc_layer = pl.kernel(
    functools.partial(_sc_body, False),
    out_type=jax.ShapeDtypeStruct((NC, NP, NH), _f32),
    mesh=_SC_MESH,
    scratch_types=(
        [pltpu.VMEM((B,), jnp.int32)] * 6
        + [pltpu.VMEM((B, TW), _f32)] * 2
        + [pltpu.VMEM((B, UW), _f32)] * 2
        + [pltpu.VMEM((B, NH), _f32)] * 2
        + [pltpu.VMEM((ROWS_PT, NH), _f32),
           pltpu.VMEM_SHARED((NP, NH), _f32)]
        + [pltpu.SemaphoreType.DMA] * 10
    ),
    compiler_params=_SC_PARAMS,
)


def kernel(x, edge_index, W1, b1, W2, b2, W3, b3,
           Wl0, U0, c0, bb0, Wl1, U1, c1, bb1, Wl2, U2, c2, bb2):
    f32 = jnp.float32

    def pad_u(u):
        return jnp.tile(u.astype(f32), (1, UW // HEADS))

    def pad_c(c):
        return jnp.tile(c.astype(f32), UW // HEADS).reshape(1, UW)

    def wls(wl, c):
        wc = jax.nn.softmax(c.astype(f32))
        return (wl.astype(f32).reshape(NH, HEADS, NH) * wc[None, :, None]).sum(axis=1)

    pad = EPW_PAD - EPW
    if pad:
        srcs = jnp.concatenate(
            [edge_index[0].astype(jnp.int32).reshape(NW, EPW),
             jnp.zeros((NW, pad), jnp.int32)], axis=1).reshape(-1)
        dsts = jnp.concatenate(
            [edge_index[1].astype(jnp.int32).reshape(NW, EPW),
             jnp.full((NW, pad), N, jnp.int32)], axis=1).reshape(-1)
    else:
        srcs = edge_index[0].astype(jnp.int32)
        dsts = edge_index[1].astype(jnp.int32)
    z32 = jnp.zeros((NP, NH), f32)
    z16 = jnp.zeros((NP, UW), f32)
    ones = jnp.ones((B, UW), f32)

    b1r = b1.reshape(1, NH).astype(f32)
    b2r = b2.reshape(1, NH // 2).astype(f32)
    b3r = b3.reshape(1, D_OUT).astype(f32)
    bb = [b.reshape(1, NH).astype(f32) for b in (bb0, bb1, bb2)]
    wl = [w.astype(f32) for w in (Wl0, Wl1, Wl2)]
    u16 = [pad_u(u) for u in (U0, U1, U2)]
    c16 = [pad_c(c) for c in (c0, c1, c2)]
    wlss = [wls(w, c) for w, c in ((Wl0, c0), (Wl1, c1), (Wl2, c2))]

    h0, t0, tu0 = _tc_pre(x.astype(f32), W1.astype(f32), b1r,
                          wl[0], u16[0], c16[0])
    acc0, deg = _sc_layer0(t0, tu0, srcs, dsts, z32, z16, ones)
    h1, t1, tu1 = _tc_mid(acc0, deg, h0, wlss[0], bb[0],
                          wl[1], u16[1], c16[1])
    acc1 = _sc_layer(t1, tu1, srcs, dsts, z32)
    h2, t2, tu2 = _tc_mid(acc1, deg, h1, wlss[1], bb[1],
                          wl[2], u16[2], c16[2])
    acc2 = _sc_layer(t2, tu2, srcs, dsts, z32)
    return _tc_post(acc2, deg, h2, wlss[2], bb[2],
                    W2.astype(f32), b2r, W3.astype(f32), b3r)
